# baseline (speedup 1.0000x reference)
import functools

import jax
import jax.numpy as jnp
from jax import lax
from jax.experimental import pallas as pl
from jax.experimental.pallas import tpu as pltpu
from jax.experimental.pallas import tpu_sc as plsc

_USERS = 80000
_N = 100000
_B = 4096
_D = 128
_H = 64
_LLM = 1536
_K = 16
_NBR = 32
_NW = 32
_BPW = _B // _NW
_SCH = 8
_BLK = 256


def _sc_gather_body(mem, uidx, iidx, gidx, nbrtab4, pc, pi, pcat, pbr, lu128,
                    users_o, items_o, neg_o, pc_o, pi_o, pcat_o, pbr_o, lu_o,
                    nsu_o, nsi_o, nsn_o,
                    idx_u, idx_it, idx_ng, idx_t,
                    rowbuf, idsrows, ids_ka, ids_kb, ids_kc,
                    acca, accb, accc, luv,
                    sem, sema, semb, semc):
    wid = lax.axis_index("s") * 2 + lax.axis_index("c")
    base = wid * _BPW
    lanes = lax.iota(jnp.int32, 16)
    nchunk = _BPW // 16
    zero16 = jnp.zeros((16,), jnp.float32)

    pltpu.sync_copy(uidx.at[pl.ds(base, _BPW)], idx_u)
    pltpu.sync_copy(iidx.at[pl.ds(base, _BPW)], idx_it)
    pltpu.sync_copy(gidx.at[pl.ds(base, _BPW)], idx_ng)

    def nbr_start(idxv, ids_k, acc, semx):
        for c in range(nchunk):
            s = pl.ds(c * 16, 16)
            idx_t[s] = lax.shift_right_logical(idxv[s], 2)
        pltpu.async_copy(nbrtab4.at[idx_t], idsrows, sem).wait()

        def compact(k, carry):
            kv = jnp.full((16,), 0, jnp.int32) + k
            for c in range(nchunk):
                s = pl.ds(c * 16, 16)
                pos = lanes + c * 16
                col = jnp.bitwise_and(idxv[s], 3) * _NBR + k
                ids16 = plsc.load_gather(idsrows, [pos, col])
                plsc.store_scatter(ids_k, [kv * _BPW + pos], ids16)
            return carry

        lax.fori_loop(0, _NBR, compact, 0)

        def zero(n, carry):
            for c in range(_D // 16):
                acc[n, pl.ds(c * 16, 16)] = zero16
            return carry

        lax.fori_loop(0, _BPW, zero, 0)

        for k in range(_NBR):
            pltpu.async_copy(mem.at[ids_k.at[pl.ds(k * _BPW, _BPW)]],
                             acc, semx, add=True)

    def nbr_drain(ids_k, acc, semx, out):
        for k in range(_NBR):
            pltpu.make_async_copy(mem.at[ids_k.at[pl.ds(k * _BPW, _BPW)]],
                                  acc, semx).wait()
        pltpu.sync_copy(acc, out.at[pl.ds(base, _BPW)])

    nbr_start(idx_u, ids_ka, acca, sema)
    nbr_start(idx_it, ids_kb, accb, semb)
    nbr_start(idx_ng, ids_kc, accc, semc)

    def grow(table, idxv, out):
        pltpu.async_copy(table.at[idxv], rowbuf, sem).wait()
        pltpu.sync_copy(rowbuf, out.at[pl.ds(base, _BPW)])

    grow(mem, idx_u, users_o)
    grow(mem, idx_it, items_o)
    grow(mem, idx_ng, neg_o)
    grow(pc, idx_u, pc_o)
    grow(pi, idx_u, pi_o)
    grow(pcat, idx_u, pcat_o)
    grow(pbr, idx_u, pbr_o)

    for c in range(nchunk):
        s = pl.ds(c * 16, 16)
        idx_t[s] = lax.shift_right_logical(idx_u[s], 7)
    pltpu.async_copy(lu128.at[idx_t], rowbuf, sem).wait()
    for c in range(nchunk):
        s = pl.ds(c * 16, 16)
        luv[s] = plsc.load_gather(
            rowbuf, [lanes + c * 16, jnp.bitwise_and(idx_u[s], 127)])
    pltpu.sync_copy(luv, lu_o.at[pl.ds(base, _BPW)])

    nbr_drain(ids_ka, acca, sema, nsu_o)
    nbr_drain(ids_kb, accb, semb, nsi_o)
    nbr_drain(ids_kc, accc, semc, nsn_o)


@functools.cache
def _sc_gather_call():
  return pl.kernel(
    _sc_gather_body,
    out_type=[
        jax.ShapeDtypeStruct((_B, _D), jnp.float32),
        jax.ShapeDtypeStruct((_B, _D), jnp.float32),
        jax.ShapeDtypeStruct((_B, _D), jnp.float32),
        jax.ShapeDtypeStruct((_B, _D), jnp.float32),
        jax.ShapeDtypeStruct((_B, _D), jnp.float32),
        jax.ShapeDtypeStruct((_B, _D), jnp.float32),
        jax.ShapeDtypeStruct((_B, _D), jnp.float32),
        jax.ShapeDtypeStruct((_B,), jnp.float32),
        jax.ShapeDtypeStruct((_B, _D), jnp.float32),
        jax.ShapeDtypeStruct((_B, _D), jnp.float32),
        jax.ShapeDtypeStruct((_B, _D), jnp.float32),
    ],
    mesh=plsc.VectorSubcoreMesh(core_axis_name="c", subcore_axis_name="s"),
    compiler_params=pltpu.CompilerParams(needs_layout_passes=False),
    scratch_types=[
        pltpu.VMEM((_BPW,), jnp.int32),
        pltpu.VMEM((_BPW,), jnp.int32),
        pltpu.VMEM((_BPW,), jnp.int32),
        pltpu.VMEM((_BPW,), jnp.int32),
        pltpu.VMEM((_BPW, _D), jnp.float32),
        pltpu.VMEM((_BPW, _D), jnp.int32),
        pltpu.VMEM((_BPW * _NBR,), jnp.int32),
        pltpu.VMEM((_BPW * _NBR,), jnp.int32),
        pltpu.VMEM((_BPW * _NBR,), jnp.int32),
        pltpu.VMEM((_BPW, _D), jnp.float32),
        pltpu.VMEM((_BPW, _D), jnp.float32),
        pltpu.VMEM((_BPW, _D), jnp.float32),
        pltpu.VMEM((_BPW,), jnp.float32),
        pltpu.SemaphoreType.DMA,
        pltpu.SemaphoreType.DMA,
        pltpu.SemaphoreType.DMA,
        pltpu.SemaphoreType.DMA,
    ],
  )


def _sc_stat_body(stat2, iidx, gidx, sti_o, stn_o,
                  idx_b, idx_t, rowbuf, sem):
    wid = lax.axis_index("s") * 2 + lax.axis_index("c")
    base = wid * _BPW
    nchunk = _BPW // 16

    for src, out in ((iidx, sti_o), (gidx, stn_o)):
        pltpu.sync_copy(src.at[pl.ds(base, _BPW)], idx_b)
        for c in range(nchunk):
            s = pl.ds(c * 16, 16)
            idx_t[s] = lax.shift_right_logical(idx_b[s] - _USERS, 1)
        pltpu.async_copy(stat2.at[idx_t], rowbuf, sem).wait()
        pltpu.sync_copy(rowbuf, out.at[pl.ds(base, _BPW)])


@functools.cache
def _sc_stat_call():
  return pl.kernel(
    _sc_stat_body,
    out_type=[
        jax.ShapeDtypeStruct((_B, _D), jnp.float32),
        jax.ShapeDtypeStruct((_B, _D), jnp.float32),
    ],
    mesh=plsc.VectorSubcoreMesh(core_axis_name="c", subcore_axis_name="s"),
    compiler_params=pltpu.CompilerParams(needs_layout_passes=False),
    scratch_types=[
        pltpu.VMEM((_BPW,), jnp.int32),
        pltpu.VMEM((_BPW,), jnp.int32),
        pltpu.VMEM((_BPW, _D), jnp.float32),
        pltpu.SemaphoreType.DMA,
    ],
  )


def _ln(x):
    m = jnp.mean(x, axis=-1, keepdims=True)
    v = jnp.mean((x - m) ** 2, axis=-1, keepdims=True)
    return (x - m) * lax.rsqrt(v + 1e-5)


def _dot(a, b):
    return jnp.dot(a, b, preferred_element_type=jnp.float32)


def _topk(x):
    vals = []
    for _ in range(_K):
        m = jnp.max(x, axis=-1, keepdims=True)
        vals.append(m)
        x = jnp.where(x == m, -jnp.inf, x)
    return jnp.concatenate(vals, axis=-1)


def _stat_body(emb, w_item, b_item, out):
    out[...] = _ln(_dot(emb[...], w_item[...]) + b_item[...])


def _feat_body(fc, fi, fcat, fbr, wc, wi, wcat, wbr, bc, bi, bcat, bbr, out):
    srcs = ((fc, wc, bc), (fi, wi, bi), (fcat, wcat, bcat), (fbr, wbr, bbr))
    for s, (f, w, b) in enumerate(srcs):
        out[:, s, :] = _ln(_dot(f[...], w[...]) + b[...])


def _main_body(sti, stn, iid, gid, ue, ie, ge, nsu, nsi, nsn,
               pcg, pig, pcatg, pbrg, lug, tsg,
               w_agg, b_agg, w_proj, w_um, b_um, w_im, b_im,
               wq, bq, wk, bk, wv, bv, wo, bo, w_m1, b_m1, w_m2, b_m2,
               um_o, im_o, nm_o):
    it_odd = jnp.bitwise_and(iid[...], 1) == 1
    ng_odd = jnp.bitwise_and(gid[...], 1) == 1
    it_static = jnp.where(it_odd, sti[:, _H:], sti[:, :_H])
    ng_static = jnp.where(ng_odd, stn[:, _H:], stn[:, :_H])

    ue_ = ue[...]
    ie_ = ie[...]
    ge_ = ge[...]
    inv = jnp.float32(1.0 / _NBR)

    def agg(e, ns):
        return jax.nn.relu(_dot(e, w_agg[: _D, :]) +
                           _dot(ns[...] * inv, w_agg[_D:, :]) + b_agg[...])

    users_nb = agg(ue_, nsu)
    items_nb = agg(ie_, nsi)
    neg_nb = agg(ge_, nsn)

    td = tsg[...] - lug[...]
    users_proj = _ln(ue_ * (1.0 + td * w_proj[...]))
    users_agg = (_dot(ue_, w_um[: _D, :]) + _dot(users_nb, w_um[_D: 2 * _D, :])
                 + _dot(users_proj, w_um[2 * _D:, :]) + b_um[...])

    profs = (_topk(pcg[...]), _topk(pig[...]), _topk(pcatg[...]),
             _topk(pbrg[...]))

    q = _dot(users_agg, wq[...]) + bq[...]
    kk = [_dot(p, wk[...]) + bk[...] for p in profs]
    vv = [_dot(p, wv[...]) + bv[...] for p in profs]
    scale = jnp.float32(1.0 / 8.0)
    lane = lax.broadcasted_iota(jnp.int32, (1, _D), 1)
    mlo = (lane < _H).astype(jnp.float32)
    mhi = jnp.float32(1.0) - mlo
    s0, s1 = [], []
    for s in range(4):
        prod = q * kk[s]
        s0.append(jnp.sum(prod * mlo, axis=-1, keepdims=True) * scale)
        s1.append(jnp.sum(prod * mhi, axis=-1, keepdims=True) * scale)
    sc0 = jnp.concatenate(s0, axis=-1)
    sc1 = jnp.concatenate(s1, axis=-1)

    def smax(x):
        m = jnp.max(x, axis=-1, keepdims=True)
        e = jnp.exp(x - m)
        return e / jnp.sum(e, axis=-1, keepdims=True)

    aw0 = smax(sc0)
    aw1 = smax(sc1)
    ao = jnp.zeros_like(q)
    for s in range(4):
        wfull = aw0[:, s:s + 1] * mlo + aw1[:, s:s + 1] * mhi
        ao = ao + wfull * vv[s]
    attn_out = _dot(ao, wo[...]) + bo[...]

    hm = jax.nn.relu(_dot(users_agg, w_m1[: _D, :]) +
                     _dot(attn_out, w_m1[_D:, :]) + b_m1[...])
    um_o[...] = _dot(hm, w_m2[...]) + b_m2[...]

    idm = (_dot(ie_[:, _H:], w_im[: _H, :]) + _dot(items_nb, w_im[_H:, :])
           + b_im[...])
    im_o[...] = jnp.concatenate([it_static, idm], axis=-1)
    ndm = (_dot(ge_[:, _H:], w_im[: _H, :]) + _dot(neg_nb, w_im[_H:, :])
           + b_im[...])
    nm_o[...] = jnp.concatenate([ng_static, ndm], axis=-1)


def _row_spec(width):
    return pl.BlockSpec((_BLK, width), lambda i: (i, 0))


def _full_spec(shape):
    nd = len(shape)
    return pl.BlockSpec(shape, lambda i, _nd=nd: (0,) * _nd)


_GRID = _B // _BLK


_SBLK = 2000


def _build_tc_calls(interpret=False):
    stat = pl.pallas_call(
        _stat_body,
        grid=(20000 // _SBLK,),
        in_specs=[
            pl.BlockSpec((_SBLK, _LLM), lambda i: (i, 0)),
            _full_spec((_LLM, _H)),
            _full_spec((1, _H)),
        ],
        out_specs=pl.BlockSpec((_SBLK, _H), lambda i: (i, 0)),
        out_shape=jax.ShapeDtypeStruct((20000, _H), jnp.float32),
        interpret=interpret,
    )
    feat = pl.pallas_call(
        _feat_body,
        grid=(_GRID,),
        in_specs=[_row_spec(_LLM)] * 4
                 + [_full_spec((_LLM, _D))] * 4
                 + [_full_spec((1, _D))] * 4,
        out_specs=pl.BlockSpec((_BLK, 4, _D), lambda i: (i, 0, 0)),
        out_shape=jax.ShapeDtypeStruct((_B, 4, _D), jnp.float32),
        interpret=interpret,
    )
    main = pl.pallas_call(
        _main_body,
        grid=(_GRID,),
        in_specs=[
            _row_spec(_D), _row_spec(_D),
            _row_spec(1), _row_spec(1),
            _row_spec(_D), _row_spec(_D), _row_spec(_D),
            _row_spec(_D), _row_spec(_D), _row_spec(_D),
            _row_spec(_D), _row_spec(_D), _row_spec(_D), _row_spec(_D),
            _row_spec(1), _row_spec(1),
            _full_spec((2 * _D, _D)), _full_spec((1, _D)),
            _full_spec((1, _D)),
            _full_spec((3 * _D, _D)), _full_spec((1, _D)),
            _full_spec((3 * _H, _H)), _full_spec((1, _H)),
            _full_spec((_D, _D)), _full_spec((1, _D)),
            _full_spec((_K, _D)), _full_spec((1, _D)),
            _full_spec((_K, _D)), _full_spec((1, _D)),
            _full_spec((_D, _D)), _full_spec((1, _D)),
            _full_spec((2 * _D, _D)), _full_spec((1, _D)),
            _full_spec((_D, _D)), _full_spec((1, _D)),
        ],
        out_specs=[_row_spec(_D)] * 3,
        out_shape=[jax.ShapeDtypeStruct((_B, _D), jnp.float32)] * 3,
        interpret=interpret,
    )
    return stat, feat, main


_stat_call, _feat_call, _main_call = _build_tc_calls()


def kernel(users_idxs_cut, items_idxs_cut, negative_items_idxs_cut,
           items_static_embeddings, crowds_features_batch,
           interests_features_batch, categories_features_batch,
           brands_features_batch, timestamps_cut, neighbor_table, params):
    p = params
    npad = (_N + _D - 1) // _D * _D
    lu128 = jnp.pad(p['nodes_last_update'], (0, npad - _N)).reshape(-1, _D)
    nbrtab4 = neighbor_table.reshape(_N * _NBR // _D, _D)

    static_t = _stat_call(items_static_embeddings, p['W_item'],
                          p['b_item'].reshape(1, _H))
    static_t2 = static_t.reshape(10000, _D)

    (ue, ie, ge, pcg, pig, pcatg, pbrg, lug, nsu, nsi, nsn) = \
        _sc_gather_call()(p['memory_nodes'], users_idxs_cut, items_idxs_cut,
                          negative_items_idxs_cut, nbrtab4,
                          p['profile_crowd'], p['profile_interest'],
                          p['profile_category'], p['profile_brand'], lu128)
    lug = lug.reshape(_B, 1)

    sti, stn = _sc_stat_call()(static_t2, items_idxs_cut,
                               negative_items_idxs_cut)

    new_profiles = _feat_call(
        crowds_features_batch, interests_features_batch,
        categories_features_batch, brands_features_batch,
        p['W_crowd'], p['W_interest'], p['W_category'], p['W_brand'],
        p['b_crowd'].reshape(1, _D), p['b_interest'].reshape(1, _D),
        p['b_category'].reshape(1, _D), p['b_brand'].reshape(1, _D))

    um, im, nm = _main_call(
        sti, stn, items_idxs_cut.reshape(_B, 1),
        negative_items_idxs_cut.reshape(_B, 1),
        ue, ie, ge, nsu, nsi, nsn, pcg, pig, pcatg, pbrg,
        lug, timestamps_cut.reshape(_B, 1),
        p['W_agg'], p['b_agg'].reshape(1, _D),
        p['w_proj'].reshape(1, _D),
        p['W_um'], p['b_um'].reshape(1, _D),
        p['W_im'], p['b_im'].reshape(1, _H),
        p['Wq'], p['bq'].reshape(1, _D),
        p['Wk'], p['bk'].reshape(1, _D),
        p['Wv'], p['bv'].reshape(1, _D),
        p['Wo'], p['bo'].reshape(1, _D),
        p['W_mrg1'], p['b_mrg1'].reshape(1, _D),
        p['W_mrg2'], p['b_mrg2'].reshape(1, _D))

    return um, im, nm, new_profiles


# speedup vs baseline: 1.1183x; 1.1183x over previous
import functools

import jax
import jax.numpy as jnp
from jax import lax
from jax.experimental import pallas as pl
from jax.experimental.pallas import tpu as pltpu
from jax.experimental.pallas import tpu_sc as plsc

_USERS = 80000
_N = 100000
_B = 4096
_D = 128
_H = 64
_LLM = 1536
_K = 16
_NBR = 32
_NW = 32
_BPW = _B // _NW
_SCH = 8
_BLK = 512


def _sc_gather_body(mem, uidx, iidx, gidx, nbrtab4, pc, pi, pcat, pbr, lu128,
                    users_o, items_o, neg_o, pc_o, pi_o, pcat_o, pbr_o, lu_o,
                    nsu_o, nsi_o, nsn_o,
                    idx_u, idx_it, idx_ng, idx_t,
                    rowbuf, idsrows, ids_ka, ids_kb, ids_kc,
                    acca, accb, accc, luv,
                    sem, sema, semb, semc):
    wid = lax.axis_index("s") * 2 + lax.axis_index("c")
    base = wid * _BPW
    lanes = lax.iota(jnp.int32, 16)
    nchunk = _BPW // 16
    zero16 = jnp.zeros((16,), jnp.float32)

    pltpu.sync_copy(uidx.at[pl.ds(base, _BPW)], idx_u)
    pltpu.sync_copy(iidx.at[pl.ds(base, _BPW)], idx_it)
    pltpu.sync_copy(gidx.at[pl.ds(base, _BPW)], idx_ng)

    def nbr_start(idxv, ids_k, acc, semx):
        for c in range(nchunk):
            s = pl.ds(c * 16, 16)
            idx_t[s] = lax.shift_right_logical(idxv[s], 2)
        pltpu.async_copy(nbrtab4.at[idx_t], idsrows, sem).wait()

        def compact(k, carry):
            kv = jnp.full((16,), 0, jnp.int32) + k
            for c in range(nchunk):
                s = pl.ds(c * 16, 16)
                pos = lanes + c * 16
                col = jnp.bitwise_and(idxv[s], 3) * _NBR + k
                ids16 = plsc.load_gather(idsrows, [pos, col])
                plsc.store_scatter(ids_k, [kv * _BPW + pos], ids16)
            return carry

        lax.fori_loop(0, _NBR, compact, 0)

        def zero(n, carry):
            for c in range(_D // 16):
                acc[n, pl.ds(c * 16, 16)] = zero16
            return carry

        lax.fori_loop(0, _BPW, zero, 0)

        for k in range(_NBR):
            pltpu.async_copy(mem.at[ids_k.at[pl.ds(k * _BPW, _BPW)]],
                             acc, semx, add=True)

    def nbr_drain(ids_k, acc, semx, out):
        for k in range(_NBR):
            pltpu.make_async_copy(mem.at[ids_k.at[pl.ds(k * _BPW, _BPW)]],
                                  acc, semx).wait()
        pltpu.sync_copy(acc, out.at[pl.ds(base, _BPW)])

    nbr_start(idx_u, ids_ka, acca, sema)
    nbr_start(idx_it, ids_kb, accb, semb)
    nbr_start(idx_ng, ids_kc, accc, semc)

    def grow(table, idxv, out):
        pltpu.async_copy(table.at[idxv], rowbuf, sem).wait()
        pltpu.sync_copy(rowbuf, out.at[pl.ds(base, _BPW)])

    grow(mem, idx_u, users_o)
    grow(mem, idx_it, items_o)
    grow(mem, idx_ng, neg_o)
    grow(pc, idx_u, pc_o)
    grow(pi, idx_u, pi_o)
    grow(pcat, idx_u, pcat_o)
    grow(pbr, idx_u, pbr_o)

    for c in range(nchunk):
        s = pl.ds(c * 16, 16)
        idx_t[s] = lax.shift_right_logical(idx_u[s], 7)
    pltpu.async_copy(lu128.at[idx_t], rowbuf, sem).wait()
    for c in range(nchunk):
        s = pl.ds(c * 16, 16)
        luv[s] = plsc.load_gather(
            rowbuf, [lanes + c * 16, jnp.bitwise_and(idx_u[s], 127)])
    pltpu.sync_copy(luv, lu_o.at[pl.ds(base, _BPW)])

    nbr_drain(ids_ka, acca, sema, nsu_o)
    nbr_drain(ids_kb, accb, semb, nsi_o)
    nbr_drain(ids_kc, accc, semc, nsn_o)


@functools.cache
def _sc_gather_call():
  return pl.kernel(
    _sc_gather_body,
    out_type=[
        jax.ShapeDtypeStruct((_B, _D), jnp.float32),
        jax.ShapeDtypeStruct((_B, _D), jnp.float32),
        jax.ShapeDtypeStruct((_B, _D), jnp.float32),
        jax.ShapeDtypeStruct((_B, _D), jnp.float32),
        jax.ShapeDtypeStruct((_B, _D), jnp.float32),
        jax.ShapeDtypeStruct((_B, _D), jnp.float32),
        jax.ShapeDtypeStruct((_B, _D), jnp.float32),
        jax.ShapeDtypeStruct((_B,), jnp.float32),
        jax.ShapeDtypeStruct((_B, _D), jnp.float32),
        jax.ShapeDtypeStruct((_B, _D), jnp.float32),
        jax.ShapeDtypeStruct((_B, _D), jnp.float32),
    ],
    mesh=plsc.VectorSubcoreMesh(core_axis_name="c", subcore_axis_name="s"),
    compiler_params=pltpu.CompilerParams(needs_layout_passes=False),
    scratch_types=[
        pltpu.VMEM((_BPW,), jnp.int32),
        pltpu.VMEM((_BPW,), jnp.int32),
        pltpu.VMEM((_BPW,), jnp.int32),
        pltpu.VMEM((_BPW,), jnp.int32),
        pltpu.VMEM((_BPW, _D), jnp.float32),
        pltpu.VMEM((_BPW, _D), jnp.int32),
        pltpu.VMEM((_BPW * _NBR,), jnp.int32),
        pltpu.VMEM((_BPW * _NBR,), jnp.int32),
        pltpu.VMEM((_BPW * _NBR,), jnp.int32),
        pltpu.VMEM((_BPW, _D), jnp.float32),
        pltpu.VMEM((_BPW, _D), jnp.float32),
        pltpu.VMEM((_BPW, _D), jnp.float32),
        pltpu.VMEM((_BPW,), jnp.float32),
        pltpu.SemaphoreType.DMA,
        pltpu.SemaphoreType.DMA,
        pltpu.SemaphoreType.DMA,
        pltpu.SemaphoreType.DMA,
    ],
  )


def _sc_stat_body(stat2, iidx, gidx, sti_o, stn_o,
                  idx_b, idx_t, rowbuf, sem):
    wid = lax.axis_index("s") * 2 + lax.axis_index("c")
    base = wid * _BPW
    nchunk = _BPW // 16

    for src, out in ((iidx, sti_o), (gidx, stn_o)):
        pltpu.sync_copy(src.at[pl.ds(base, _BPW)], idx_b)
        for c in range(nchunk):
            s = pl.ds(c * 16, 16)
            idx_t[s] = lax.shift_right_logical(idx_b[s] - _USERS, 1)
        pltpu.async_copy(stat2.at[idx_t], rowbuf, sem).wait()
        pltpu.sync_copy(rowbuf, out.at[pl.ds(base, _BPW)])


@functools.cache
def _sc_stat_call():
  return pl.kernel(
    _sc_stat_body,
    out_type=[
        jax.ShapeDtypeStruct((_B, _D), jnp.float32),
        jax.ShapeDtypeStruct((_B, _D), jnp.float32),
    ],
    mesh=plsc.VectorSubcoreMesh(core_axis_name="c", subcore_axis_name="s"),
    compiler_params=pltpu.CompilerParams(needs_layout_passes=False),
    scratch_types=[
        pltpu.VMEM((_BPW,), jnp.int32),
        pltpu.VMEM((_BPW,), jnp.int32),
        pltpu.VMEM((_BPW, _D), jnp.float32),
        pltpu.SemaphoreType.DMA,
    ],
  )


def _ln(x):
    m = jnp.mean(x, axis=-1, keepdims=True)
    v = jnp.mean((x - m) ** 2, axis=-1, keepdims=True)
    return (x - m) * lax.rsqrt(v + 1e-5)


def _dot(a, b):
    return jnp.dot(a, b, preferred_element_type=jnp.float32)


def _topk(x):
    vals = []
    for _ in range(_K):
        m = jnp.max(x, axis=-1, keepdims=True)
        vals.append(m)
        x = jnp.where(x == m, -jnp.inf, x)
    return jnp.concatenate(vals, axis=-1)


def _stat_body(emb, w_item, b_item, out):
    out[...] = _ln(_dot(emb[...], w_item[...]) + b_item[...])


def _feat_body(fc, fi, fcat, fbr, wc, wi, wcat, wbr, bc, bi, bcat, bbr, out):
    srcs = ((fc, wc, bc), (fi, wi, bi), (fcat, wcat, bcat), (fbr, wbr, bbr))
    for s, (f, w, b) in enumerate(srcs):
        out[:, s, :] = _ln(_dot(f[...], w[...]) + b[...])


def _main_body(sti, stn, iid, gid, ue, ie, ge, nsu, nsi, nsn,
               pcg, pig, pcatg, pbrg, lug, tsg,
               w_agg, b_agg, w_proj, w_um, b_um, w_im, b_im,
               wq, bq, wk, bk, wv, bv, wo, bo, w_m1, b_m1, w_m2, b_m2,
               um_o, im_o, nm_o):
    it_odd = jnp.bitwise_and(iid[...], 1) == 1
    ng_odd = jnp.bitwise_and(gid[...], 1) == 1
    it_static = jnp.where(it_odd, sti[:, _H:], sti[:, :_H])
    ng_static = jnp.where(ng_odd, stn[:, _H:], stn[:, :_H])

    ue_ = ue[...]
    ie_ = ie[...]
    ge_ = ge[...]
    inv = jnp.float32(1.0 / _NBR)

    def agg(e, ns):
        return jax.nn.relu(_dot(e, w_agg[: _D, :]) +
                           _dot(ns[...] * inv, w_agg[_D:, :]) + b_agg[...])

    users_nb = agg(ue_, nsu)
    items_nb = agg(ie_, nsi)
    neg_nb = agg(ge_, nsn)

    td = tsg[...] - lug[...]
    users_proj = _ln(ue_ * (1.0 + td * w_proj[...]))
    users_agg = (_dot(ue_, w_um[: _D, :]) + _dot(users_nb, w_um[_D: 2 * _D, :])
                 + _dot(users_proj, w_um[2 * _D:, :]) + b_um[...])

    profs = (_topk(pcg[...]), _topk(pig[...]), _topk(pcatg[...]),
             _topk(pbrg[...]))

    q = _dot(users_agg, wq[...]) + bq[...]
    kk = [_dot(p, wk[...]) + bk[...] for p in profs]
    vv = [_dot(p, wv[...]) + bv[...] for p in profs]
    scale = jnp.float32(1.0 / 8.0)
    lane = lax.broadcasted_iota(jnp.int32, (1, _D), 1)
    mlo = (lane < _H).astype(jnp.float32)
    mhi = jnp.float32(1.0) - mlo
    s0, s1 = [], []
    for s in range(4):
        prod = q * kk[s]
        s0.append(jnp.sum(prod * mlo, axis=-1, keepdims=True) * scale)
        s1.append(jnp.sum(prod * mhi, axis=-1, keepdims=True) * scale)
    sc0 = jnp.concatenate(s0, axis=-1)
    sc1 = jnp.concatenate(s1, axis=-1)

    def smax(x):
        m = jnp.max(x, axis=-1, keepdims=True)
        e = jnp.exp(x - m)
        return e / jnp.sum(e, axis=-1, keepdims=True)

    aw0 = smax(sc0)
    aw1 = smax(sc1)
    ao = jnp.zeros_like(q)
    for s in range(4):
        wfull = aw0[:, s:s + 1] * mlo + aw1[:, s:s + 1] * mhi
        ao = ao + wfull * vv[s]
    attn_out = _dot(ao, wo[...]) + bo[...]

    hm = jax.nn.relu(_dot(users_agg, w_m1[: _D, :]) +
                     _dot(attn_out, w_m1[_D:, :]) + b_m1[...])
    um_o[...] = _dot(hm, w_m2[...]) + b_m2[...]

    idm = (_dot(ie_[:, _H:], w_im[: _H, :]) + _dot(items_nb, w_im[_H:, :])
           + b_im[...])
    im_o[...] = jnp.concatenate([it_static, idm], axis=-1)
    ndm = (_dot(ge_[:, _H:], w_im[: _H, :]) + _dot(neg_nb, w_im[_H:, :])
           + b_im[...])
    nm_o[...] = jnp.concatenate([ng_static, ndm], axis=-1)


def _row_spec(width):
    return pl.BlockSpec((_BLK, width), lambda i: (i, 0))


def _full_spec(shape):
    nd = len(shape)
    return pl.BlockSpec(shape, lambda i, _nd=nd: (0,) * _nd)


_GRID = _B // _BLK


_SBLK = 2000


def _build_tc_calls(interpret=False):
    stat = pl.pallas_call(
        _stat_body,
        grid=(20000 // _SBLK,),
        in_specs=[
            pl.BlockSpec((_SBLK, _LLM), lambda i: (i, 0)),
            _full_spec((_LLM, _H)),
            _full_spec((1, _H)),
        ],
        out_specs=pl.BlockSpec((_SBLK, _H), lambda i: (i, 0)),
        out_shape=jax.ShapeDtypeStruct((20000, _H), jnp.float32),
        interpret=interpret,
    )
    feat = pl.pallas_call(
        _feat_body,
        grid=(_GRID,),
        in_specs=[_row_spec(_LLM)] * 4
                 + [_full_spec((_LLM, _D))] * 4
                 + [_full_spec((1, _D))] * 4,
        out_specs=pl.BlockSpec((_BLK, 4, _D), lambda i: (i, 0, 0)),
        out_shape=jax.ShapeDtypeStruct((_B, 4, _D), jnp.float32),
        interpret=interpret,
    )
    main = pl.pallas_call(
        _main_body,
        grid=(_GRID,),
        in_specs=[
            _row_spec(_D), _row_spec(_D),
            _row_spec(1), _row_spec(1),
            _row_spec(_D), _row_spec(_D), _row_spec(_D),
            _row_spec(_D), _row_spec(_D), _row_spec(_D),
            _row_spec(_D), _row_spec(_D), _row_spec(_D), _row_spec(_D),
            _row_spec(1), _row_spec(1),
            _full_spec((2 * _D, _D)), _full_spec((1, _D)),
            _full_spec((1, _D)),
            _full_spec((3 * _D, _D)), _full_spec((1, _D)),
            _full_spec((3 * _H, _H)), _full_spec((1, _H)),
            _full_spec((_D, _D)), _full_spec((1, _D)),
            _full_spec((_K, _D)), _full_spec((1, _D)),
            _full_spec((_K, _D)), _full_spec((1, _D)),
            _full_spec((_D, _D)), _full_spec((1, _D)),
            _full_spec((2 * _D, _D)), _full_spec((1, _D)),
            _full_spec((_D, _D)), _full_spec((1, _D)),
        ],
        out_specs=[_row_spec(_D)] * 3,
        out_shape=[jax.ShapeDtypeStruct((_B, _D), jnp.float32)] * 3,
        interpret=interpret,
    )
    return stat, feat, main


_stat_call, _feat_call, _main_call = _build_tc_calls()


def kernel(users_idxs_cut, items_idxs_cut, negative_items_idxs_cut,
           items_static_embeddings, crowds_features_batch,
           interests_features_batch, categories_features_batch,
           brands_features_batch, timestamps_cut, neighbor_table, params):
    p = params
    npad = (_N + _D - 1) // _D * _D
    lu128 = jnp.pad(p['nodes_last_update'], (0, npad - _N)).reshape(-1, _D)
    nbrtab4 = neighbor_table.reshape(_N * _NBR // _D, _D)

    static_t = _stat_call(items_static_embeddings, p['W_item'],
                          p['b_item'].reshape(1, _H))
    static_t2 = static_t.reshape(10000, _D)

    (ue, ie, ge, pcg, pig, pcatg, pbrg, lug, nsu, nsi, nsn) = \
        _sc_gather_call()(p['memory_nodes'], users_idxs_cut, items_idxs_cut,
                          negative_items_idxs_cut, nbrtab4,
                          p['profile_crowd'], p['profile_interest'],
                          p['profile_category'], p['profile_brand'], lu128)
    lug = lug.reshape(_B, 1)

    sti, stn = _sc_stat_call()(static_t2, items_idxs_cut,
                               negative_items_idxs_cut)

    new_profiles = _feat_call(
        crowds_features_batch, interests_features_batch,
        categories_features_batch, brands_features_batch,
        p['W_crowd'], p['W_interest'], p['W_category'], p['W_brand'],
        p['b_crowd'].reshape(1, _D), p['b_interest'].reshape(1, _D),
        p['b_category'].reshape(1, _D), p['b_brand'].reshape(1, _D))

    um, im, nm = _main_call(
        sti, stn, items_idxs_cut.reshape(_B, 1),
        negative_items_idxs_cut.reshape(_B, 1),
        ue, ie, ge, nsu, nsi, nsn, pcg, pig, pcatg, pbrg,
        lug, timestamps_cut.reshape(_B, 1),
        p['W_agg'], p['b_agg'].reshape(1, _D),
        p['w_proj'].reshape(1, _D),
        p['W_um'], p['b_um'].reshape(1, _D),
        p['W_im'], p['b_im'].reshape(1, _H),
        p['Wq'], p['bq'].reshape(1, _D),
        p['Wk'], p['bk'].reshape(1, _D),
        p['Wv'], p['bv'].reshape(1, _D),
        p['Wo'], p['bo'].reshape(1, _D),
        p['W_mrg1'], p['b_mrg1'].reshape(1, _D),
        p['W_mrg2'], p['b_mrg2'].reshape(1, _D))

    return um, im, nm, new_profiles
